# Initial kernel scaffold; baseline (speedup 1.0000x reference)
import jax
import jax.numpy as jnp
from jax.experimental import pallas as pl


def kernel(pos, batch, c1_W1, c1_b1, c1_g1, c1_be1, c1_W2, c1_b2, c1_g2, c1_be2, c1_W3, c1_b3, c2_W, c2_b, l1_W, l1_b, m1_W, m1_b, m2_W, m2_b, h_W, h_b):
    raise NotImplementedError("write your pallas kernel here")


# speedup vs baseline: 5.6094x; 5.6094x over previous
import functools

import jax
import jax.numpy as jnp
from jax import lax
from jax.experimental import pallas as pl
from jax.experimental.pallas import tpu as pltpu
from jax.experimental.pallas import tpu_sc as plsc

N = 8192
KNN = 20
KPAD = 32
NG = 16
R = 256
E = N * KNN
EPS = 1e-5


def _bf(x):
    return x.astype(jnp.bfloat16)


def _dot(a, b):
    return jnp.dot(_bf(a), _bf(b), preferred_element_type=jnp.float32)


def _vec(v):
    return jnp.reshape(v, (1, -1))


def _sq_body(x_ref, sq_ref):
    x = x_ref[...]
    sq_ref[...] = jnp.sum(x * x, axis=1, keepdims=True)


def _sqnorm(x, F):
    return pl.pallas_call(
        _sq_body,
        grid=(N // R,),
        in_specs=[pl.BlockSpec((R, F), lambda i: (i, 0))],
        out_specs=pl.BlockSpec((R, 1), lambda i: (i, 0)),
        out_shape=jax.ShapeDtypeStruct((N, 1), jnp.float32),
    )(x)


def _knn_body(xr_ref, sqr_ref, br_ref, xf_ref, sqf_ref, bf_ref, idx_ref):
    i0 = pl.program_id(0) * R
    d = sqr_ref[...] + sqf_ref[...] - 2.0 * lax.dot_general(
        _bf(xr_ref[...]), _bf(xf_ref[...]), (((1,), (1,)), ((), ())),
        preferred_element_type=jnp.float32)
    colid = lax.broadcasted_iota(jnp.int32, (R, N), 1)
    rowid = i0 + lax.broadcasted_iota(jnp.int32, (R, N), 0)
    same = (br_ref[...] == bf_ref[...]) & (colid != rowid)
    d = jnp.where(same, d, 1e37)
    cols = []
    for _t in range(KNN):
        m = jnp.min(d, axis=1, keepdims=True)
        am = jnp.min(jnp.where(d <= m, colid, N), axis=1, keepdims=True)
        cols.append(am)
        d = jnp.where(colid == am, 2e38, d)
    for _t in range(KPAD - KNN):
        cols.append(cols[0])
    idx_ref[...] = jnp.concatenate(cols, axis=1)


def _knn(x, sq_col, bat_col, sq_row, bat_row, F):
    return pl.pallas_call(
        _knn_body,
        grid=(N // R,),
        in_specs=[
            pl.BlockSpec((R, F), lambda i: (i, 0)),
            pl.BlockSpec((R, 1), lambda i: (i, 0)),
            pl.BlockSpec((R, 1), lambda i: (i, 0)),
            pl.BlockSpec((N, F), lambda i: (0, 0)),
            pl.BlockSpec((1, N), lambda i: (0, 0)),
            pl.BlockSpec((1, N), lambda i: (0, 0)),
        ],
        out_specs=pl.BlockSpec((R, KPAD), lambda i: (i, 0)),
        out_shape=jax.ShapeDtypeStruct((N, KPAD), jnp.int32),
    )(x, sq_col, bat_col, x, sq_row, bat_row)


def _gather_rows(table, idx, D, C):
    info = plsc.get_sparse_core_info()
    nw = info.num_cores * info.num_subcores
    bpw = E // nw
    nchunk = bpw // C
    mesh = plsc.VectorSubcoreMesh(core_axis_name="c", subcore_axis_name="s")

    @functools.partial(
        pl.kernel, mesh=mesh,
        out_type=jax.ShapeDtypeStruct((E, D), jnp.float32),
        scratch_types=[
            pltpu.VMEM((C,), jnp.int32),
            pltpu.VMEM((C, D), jnp.float32),
            pltpu.SemaphoreType.DMA,
        ],
    )
    def k(table_hbm, idx_hbm, out_hbm, idx_v, rows_v, sem):
        wid = lax.axis_index("s") * info.num_cores + lax.axis_index("c")
        base = wid * bpw
        for c in range(nchunk):
            off = base + c * C
            pltpu.sync_copy(idx_hbm.at[pl.ds(off, C)], idx_v)
            pltpu.async_copy(table_hbm.at[idx_v], rows_v, sem).wait()
            pltpu.sync_copy(rows_v, out_hbm.at[pl.ds(off, C)])

    return k(table, idx)


def _h1(pos_ref, bg_ref, wa_ref, wb_ref, b1_ref):
    xi = pos_ref[...]
    xia = _dot(xi, wa_ref[...]) + b1_ref[...]
    dx = bg_ref[...][:, :, 0:8] - xi[:, None, :]
    hb = _dot(jnp.reshape(dx, (R * KNN, 8)), wb_ref[...])
    return jnp.reshape(xia[:, None, :] + jnp.reshape(hb, (R, KNN, 64)),
                       (R * KNN, 64))


def _stats_from(st, cnt):
    mu = st[0:1, :] / cnt
    var = st[1:2, :] / cnt - mu * mu
    inv = lax.rsqrt(var + EPS)
    return mu, inv


def _acc_stats(st_ref, h):
    s = jnp.sum(h, axis=0, keepdims=True)
    s2 = jnp.sum(h * h, axis=0, keepdims=True)

    @pl.when(pl.program_id(0) == 0)
    def _():
        st_ref[...] = jnp.zeros_like(st_ref)

    st_ref[...] += jnp.concatenate([s, s2], axis=0)


def _e1_body(pos_ref, bg_ref, wa_ref, wb_ref, b1_ref, st_ref):
    _acc_stats(st_ref, _h1(pos_ref, bg_ref, wa_ref, wb_ref, b1_ref))


def _e2_body(pos_ref, bg_ref, wa_ref, wb_ref, b1_ref, st1_ref, g1_ref,
             be1_ref, w2_ref, b2_ref, st_ref):
    h = _h1(pos_ref, bg_ref, wa_ref, wb_ref, b1_ref)
    mu, inv = _stats_from(st1_ref[...], float(E))
    h = jnp.maximum((h - mu) * inv * g1_ref[...] + be1_ref[...], 0.0)
    h2 = _dot(h, w2_ref[...]) + b2_ref[...]
    _acc_stats(st_ref, h2)


def _e3_body(pos_ref, bg_ref, wa_ref, wb_ref, b1_ref, st1_ref, g1_ref,
             be1_ref, w2_ref, b2_ref, st2_ref, g2_ref, be2_ref, w3_ref,
             b3_ref, x1_ref, sq_ref):
    h = _h1(pos_ref, bg_ref, wa_ref, wb_ref, b1_ref)
    mu1, inv1 = _stats_from(st1_ref[...], float(E))
    h = jnp.maximum((h - mu1) * inv1 * g1_ref[...] + be1_ref[...], 0.0)
    h = _dot(h, w2_ref[...]) + b2_ref[...]
    mu2, inv2 = _stats_from(st2_ref[...], float(E))
    h = jnp.maximum((h - mu2) * inv2 * g2_ref[...] + be2_ref[...], 0.0)
    h = _dot(h, w3_ref[...]) + b3_ref[...]
    x1 = jnp.max(jnp.reshape(h, (R, KNN, 64)), axis=1)
    x1_ref[...] = x1
    sq_ref[...] = jnp.sum(x1 * x1, axis=1, keepdims=True)


def _conv1(posp, bg1, wa, wb, b1, g1, be1, w2, b2, g2, be2, w3, b3):
    grid = (N // R,)
    pos_spec = pl.BlockSpec((R, 8), lambda i: (i, 0))
    bg_spec = pl.BlockSpec((R, KNN, 128), lambda i: (i, 0, 0))
    st_spec = pl.BlockSpec((2, 64), lambda i: (0, 0))
    w8_spec = pl.BlockSpec((8, 64), lambda i: (0, 0))
    w_spec = pl.BlockSpec((64, 64), lambda i: (0, 0))
    v_spec = pl.BlockSpec((1, 64), lambda i: (0, 0))
    st_shape = jax.ShapeDtypeStruct((2, 64), jnp.float32)

    st1 = pl.pallas_call(
        _e1_body, grid=grid,
        in_specs=[pos_spec, bg_spec, w8_spec, w8_spec, v_spec],
        out_specs=st_spec, out_shape=st_shape)(posp, bg1, wa, wb, b1)
    st2 = pl.pallas_call(
        _e2_body, grid=grid,
        in_specs=[pos_spec, bg_spec, w8_spec, w8_spec, v_spec,
                  st_spec, v_spec, v_spec, w_spec, v_spec],
        out_specs=st_spec, out_shape=st_shape)(
            posp, bg1, wa, wb, b1, st1, g1, be1, w2, b2)
    x1, sq1 = pl.pallas_call(
        _e3_body, grid=grid,
        in_specs=[pos_spec, bg_spec, w8_spec, w8_spec, v_spec,
                  st_spec, v_spec, v_spec, w_spec, v_spec,
                  st_spec, v_spec, v_spec, w_spec, v_spec],
        out_specs=[pl.BlockSpec((R, 64), lambda i: (i, 0)),
                   pl.BlockSpec((R, 1), lambda i: (i, 0))],
        out_shape=[jax.ShapeDtypeStruct((N, 64), jnp.float32),
                   jax.ShapeDtypeStruct((N, 1), jnp.float32)])(
            posp, bg1, wa, wb, b1, st1, g1, be1, w2, b2,
            st2, g2, be2, w3, b3)
    return x1, sq1


def _l1_body(x1_ref, bg2_ref, bat_ref, c2w_ref, c2b_ref, lw_ref, lb_ref,
             m1w_ref, m1b_ref, m2w_ref, m2b_ref, hw_ref, hb_ref,
             out_ref, acc_ref):
    g = pl.program_id(0)
    x1 = x1_ref[...]
    xia = _dot(x1, c2w_ref[0:64, :]) + c2b_ref[...]
    dx = bg2_ref[...][:, :, 0:64] - x1[:, None, :]
    hb = _dot(jnp.reshape(dx, (R * KNN, 64)), c2w_ref[64:128, :])
    x2 = xia + jnp.max(jnp.reshape(hb, (R, KNN, 128)), axis=1)
    y = (_dot(x1, lw_ref[0:64, :]) + _dot(x2, lw_ref[64:192, :])
         + lb_ref[...])

    @pl.when(g == 0)
    def _():
        acc_ref[...] = jnp.full_like(acc_ref, -jnp.inf)

    bat = bat_ref[...]
    for s in range(NG):
        my = jnp.max(jnp.where(bat == s, y, -jnp.inf), axis=0, keepdims=True)
        acc_ref[s:s + 1, :] = jnp.maximum(acc_ref[s:s + 1, :], my)

    @pl.when(g == (N // R) - 1)
    def _():
        t = _dot(acc_ref[...], m1w_ref[...]) + m1b_ref[...]
        t = _dot(t, m2w_ref[...]) + m2b_ref[...]
        out_ref[...] = _dot(t, hw_ref[...]) + hb_ref[...]


def _l1(x1, bg2, bat_col, c2_W, c2_b, l1_W, l1_b, m1_W, m1_b, m2_W, m2_b,
        h_W, h_b):
    full = lambda r, c: pl.BlockSpec((r, c), lambda i: (0, 0))
    return pl.pallas_call(
        _l1_body,
        grid=(N // R,),
        in_specs=[
            pl.BlockSpec((R, 64), lambda i: (i, 0)),
            pl.BlockSpec((R, KNN, 128), lambda i: (i, 0, 0)),
            pl.BlockSpec((R, 1), lambda i: (i, 0)),
            full(128, 128), full(1, 128),
            full(192, 1024), full(1, 1024),
            full(1024, 512), full(1, 512),
            full(512, 256), full(1, 256),
            full(256, 40), full(1, 40),
        ],
        out_specs=full(NG, 40),
        out_shape=jax.ShapeDtypeStruct((NG, 40), jnp.float32),
        scratch_shapes=[pltpu.VMEM((NG, 1024), jnp.float32)],
    )(x1, bg2, bat_col, c2_W, _vec(c2_b), l1_W, _vec(l1_b), m1_W,
      _vec(m1_b), m2_W, _vec(m2_b), h_W, _vec(h_b))


def kernel(pos, batch, c1_W1, c1_b1, c1_g1, c1_be1, c1_W2, c1_b2, c1_g2,
           c1_be2, c1_W3, c1_b3, c2_W, c2_b, l1_W, l1_b, m1_W, m1_b,
           m2_W, m2_b, h_W, h_b):
    bat = batch.astype(jnp.int32)
    bat_col = jnp.reshape(bat, (N, 1))
    bat_row = jnp.reshape(bat, (1, N))

    posp = jnp.concatenate([pos, jnp.zeros((N, 5), jnp.float32)], axis=1)
    post = jnp.concatenate([pos, jnp.zeros((N, 125), jnp.float32)], axis=1)
    zpad = jnp.zeros((5, 64), jnp.float32)
    wa = jnp.concatenate([c1_W1[0:3, :], zpad], axis=0)
    wb = jnp.concatenate([c1_W1[3:6, :], zpad], axis=0)

    sqp = _sqnorm(posp, 8)
    idx1 = _knn(posp, sqp, bat_col, jnp.reshape(sqp, (1, N)), bat_row, 8)
    idx1f = jnp.reshape(idx1[:, :KNN], (E,))
    bg1 = jnp.reshape(_gather_rows(post, idx1f, 128, 256), (N, KNN, 128))

    x1, sq1 = _conv1(posp, bg1, wa, wb, _vec(c1_b1), _vec(c1_g1),
                     _vec(c1_be1), c1_W2, _vec(c1_b2), _vec(c1_g2),
                     _vec(c1_be2), c1_W3, _vec(c1_b3))

    x1t = jnp.concatenate([x1, jnp.zeros((N, 64), jnp.float32)], axis=1)
    idx2 = _knn(x1, sq1, bat_col, jnp.reshape(sq1, (1, N)), bat_row, 64)
    idx2f = jnp.reshape(idx2[:, :KNN], (E,))
    bg2 = jnp.reshape(_gather_rows(x1t, idx2f, 128, 256), (N, KNN, 128))

    return _l1(x1, bg2, bat_col, c2_W, c2_b, l1_W, l1_b, m1_W, m1_b,
               m2_W, m2_b, h_W, h_b)


# speedup vs baseline: 5.6156x; 1.0011x over previous
import functools

import jax
import jax.numpy as jnp
from jax import lax
from jax.experimental import pallas as pl
from jax.experimental.pallas import tpu as pltpu
from jax.experimental.pallas import tpu_sc as plsc

N = 8192
KNN = 20
KPAD = 32
NG = 16
R = 256
E = N * KNN
EPS = 1e-5


def _bf(x):
    return x.astype(jnp.bfloat16)


def _dot(a, b):
    return jnp.dot(_bf(a), _bf(b), preferred_element_type=jnp.float32)


def _vec(v):
    return jnp.reshape(v, (1, -1))


def _sq_body(x_ref, sq_ref):
    x = x_ref[...]
    sq_ref[...] = jnp.sum(x * x, axis=1, keepdims=True)


def _sqnorm(x, F):
    return pl.pallas_call(
        _sq_body,
        grid=(N // R,),
        in_specs=[pl.BlockSpec((R, F), lambda i: (i, 0))],
        out_specs=pl.BlockSpec((R, 1), lambda i: (i, 0)),
        out_shape=jax.ShapeDtypeStruct((N, 1), jnp.float32),
    )(x)


def _knn_body(xr_ref, sqr_ref, br_ref, xf_ref, sqf_ref, bf_ref, idx_ref):
    i0 = pl.program_id(0) * R
    d = sqr_ref[...] + sqf_ref[...] - 2.0 * lax.dot_general(
        _bf(xr_ref[...]), _bf(xf_ref[...]), (((1,), (1,)), ((), ())),
        preferred_element_type=jnp.float32)
    colid = lax.broadcasted_iota(jnp.int32, (R, N), 1)
    rowid = i0 + lax.broadcasted_iota(jnp.int32, (R, N), 0)
    same = (br_ref[...] == bf_ref[...]) & (colid != rowid)
    d = jnp.where(same, d, 1e37)
    cols = []
    for _t in range(KNN):
        m = jnp.min(d, axis=1, keepdims=True)
        am = jnp.min(jnp.where(d <= m, colid, N), axis=1, keepdims=True)
        cols.append(am)
        d = jnp.where(colid == am, 2e38, d)
    for _t in range(KPAD - KNN):
        cols.append(cols[0])
    idx_ref[...] = jnp.concatenate(cols, axis=1)


def _knn(x, sq_col, bat_col, sq_row, bat_row, F):
    return pl.pallas_call(
        _knn_body,
        grid=(N // R,),
        in_specs=[
            pl.BlockSpec((R, F), lambda i: (i, 0)),
            pl.BlockSpec((R, 1), lambda i: (i, 0)),
            pl.BlockSpec((R, 1), lambda i: (i, 0)),
            pl.BlockSpec((N, F), lambda i: (0, 0)),
            pl.BlockSpec((1, N), lambda i: (0, 0)),
            pl.BlockSpec((1, N), lambda i: (0, 0)),
        ],
        out_specs=pl.BlockSpec((R, KPAD), lambda i: (i, 0)),
        out_shape=jax.ShapeDtypeStruct((N, KPAD), jnp.int32),
        compiler_params=pltpu.CompilerParams(
            dimension_semantics=("parallel",)),
    )(x, sq_col, bat_col, x, sq_row, bat_row)


def _gather_rows(table, idx, D, C):
    info = plsc.get_sparse_core_info()
    nw = info.num_cores * info.num_subcores
    bpw = E // nw
    nchunk = bpw // C
    mesh = plsc.VectorSubcoreMesh(core_axis_name="c", subcore_axis_name="s")

    @functools.partial(
        pl.kernel, mesh=mesh,
        out_type=jax.ShapeDtypeStruct((E, D), jnp.float32),
        scratch_types=[
            pltpu.VMEM((C,), jnp.int32),
            pltpu.VMEM((C, D), jnp.float32),
            pltpu.SemaphoreType.DMA,
        ],
    )
    def k(table_hbm, idx_hbm, out_hbm, idx_v, rows_v, sem):
        wid = lax.axis_index("s") * info.num_cores + lax.axis_index("c")
        base = wid * bpw
        for c in range(nchunk):
            off = base + c * C
            pltpu.sync_copy(idx_hbm.at[pl.ds(off, C)], idx_v)
            pltpu.async_copy(table_hbm.at[idx_v], rows_v, sem).wait()
            pltpu.sync_copy(rows_v, out_hbm.at[pl.ds(off, C)])

    return k(table, idx)


def _h1(pos_ref, bg_ref, wa_ref, wb_ref, b1_ref):
    xi = pos_ref[...]
    xia = _dot(xi, wa_ref[...]) + b1_ref[...]
    dx = bg_ref[...][:, :, 0:8] - xi[:, None, :]
    hb = _dot(jnp.reshape(dx, (R * KNN, 8)), wb_ref[...])
    return jnp.reshape(xia[:, None, :] + jnp.reshape(hb, (R, KNN, 64)),
                       (R * KNN, 64))


def _stats_from(st, cnt):
    mu = st[0:1, :] / cnt
    var = st[1:2, :] / cnt - mu * mu
    inv = lax.rsqrt(var + EPS)
    return mu, inv


def _acc_stats(st_ref, h):
    s = jnp.sum(h, axis=0, keepdims=True)
    s2 = jnp.sum(h * h, axis=0, keepdims=True)

    @pl.when(pl.program_id(0) == 0)
    def _():
        st_ref[...] = jnp.zeros_like(st_ref)

    st_ref[...] += jnp.concatenate([s, s2], axis=0)


def _e1_body(pos_ref, bg_ref, wa_ref, wb_ref, b1_ref, st_ref):
    _acc_stats(st_ref, _h1(pos_ref, bg_ref, wa_ref, wb_ref, b1_ref))


def _e2_body(pos_ref, bg_ref, wa_ref, wb_ref, b1_ref, st1_ref, g1_ref,
             be1_ref, w2_ref, b2_ref, st_ref):
    h = _h1(pos_ref, bg_ref, wa_ref, wb_ref, b1_ref)
    mu, inv = _stats_from(st1_ref[...], float(E))
    h = jnp.maximum((h - mu) * inv * g1_ref[...] + be1_ref[...], 0.0)
    h2 = _dot(h, w2_ref[...]) + b2_ref[...]
    _acc_stats(st_ref, h2)


def _e3_body(pos_ref, bg_ref, wa_ref, wb_ref, b1_ref, st1_ref, g1_ref,
             be1_ref, w2_ref, b2_ref, st2_ref, g2_ref, be2_ref, w3_ref,
             b3_ref, x1_ref, sq_ref):
    h = _h1(pos_ref, bg_ref, wa_ref, wb_ref, b1_ref)
    mu1, inv1 = _stats_from(st1_ref[...], float(E))
    h = jnp.maximum((h - mu1) * inv1 * g1_ref[...] + be1_ref[...], 0.0)
    h = _dot(h, w2_ref[...]) + b2_ref[...]
    mu2, inv2 = _stats_from(st2_ref[...], float(E))
    h = jnp.maximum((h - mu2) * inv2 * g2_ref[...] + be2_ref[...], 0.0)
    h = _dot(h, w3_ref[...]) + b3_ref[...]
    x1 = jnp.max(jnp.reshape(h, (R, KNN, 64)), axis=1)
    x1_ref[...] = x1
    sq_ref[...] = jnp.sum(x1 * x1, axis=1, keepdims=True)


def _conv1(posp, bg1, wa, wb, b1, g1, be1, w2, b2, g2, be2, w3, b3):
    grid = (N // R,)
    pos_spec = pl.BlockSpec((R, 8), lambda i: (i, 0))
    bg_spec = pl.BlockSpec((R, KNN, 128), lambda i: (i, 0, 0))
    st_spec = pl.BlockSpec((2, 64), lambda i: (0, 0))
    w8_spec = pl.BlockSpec((8, 64), lambda i: (0, 0))
    w_spec = pl.BlockSpec((64, 64), lambda i: (0, 0))
    v_spec = pl.BlockSpec((1, 64), lambda i: (0, 0))
    st_shape = jax.ShapeDtypeStruct((2, 64), jnp.float32)

    st1 = pl.pallas_call(
        _e1_body, grid=grid,
        in_specs=[pos_spec, bg_spec, w8_spec, w8_spec, v_spec],
        out_specs=st_spec, out_shape=st_shape)(posp, bg1, wa, wb, b1)
    st2 = pl.pallas_call(
        _e2_body, grid=grid,
        in_specs=[pos_spec, bg_spec, w8_spec, w8_spec, v_spec,
                  st_spec, v_spec, v_spec, w_spec, v_spec],
        out_specs=st_spec, out_shape=st_shape)(
            posp, bg1, wa, wb, b1, st1, g1, be1, w2, b2)
    x1, sq1 = pl.pallas_call(
        _e3_body, grid=grid,
        in_specs=[pos_spec, bg_spec, w8_spec, w8_spec, v_spec,
                  st_spec, v_spec, v_spec, w_spec, v_spec,
                  st_spec, v_spec, v_spec, w_spec, v_spec],
        out_specs=[pl.BlockSpec((R, 64), lambda i: (i, 0)),
                   pl.BlockSpec((R, 1), lambda i: (i, 0))],
        out_shape=[jax.ShapeDtypeStruct((N, 64), jnp.float32),
                   jax.ShapeDtypeStruct((N, 1), jnp.float32)])(
            posp, bg1, wa, wb, b1, st1, g1, be1, w2, b2,
            st2, g2, be2, w3, b3)
    return x1, sq1


def _l1_body(x1_ref, bg2_ref, bat_ref, c2w_ref, c2b_ref, lw_ref, lb_ref,
             m1w_ref, m1b_ref, m2w_ref, m2b_ref, hw_ref, hb_ref,
             out_ref, acc_ref):
    g = pl.program_id(0)
    x1 = x1_ref[...]
    xia = _dot(x1, c2w_ref[0:64, :]) + c2b_ref[...]
    dx = bg2_ref[...][:, :, 0:64] - x1[:, None, :]
    hb = _dot(jnp.reshape(dx, (R * KNN, 64)), c2w_ref[64:128, :])
    x2 = xia + jnp.max(jnp.reshape(hb, (R, KNN, 128)), axis=1)
    y = (_dot(x1, lw_ref[0:64, :]) + _dot(x2, lw_ref[64:192, :])
         + lb_ref[...])

    @pl.when(g == 0)
    def _():
        acc_ref[...] = jnp.full_like(acc_ref, -jnp.inf)

    bat = bat_ref[...]
    for s in range(NG):
        my = jnp.max(jnp.where(bat == s, y, -jnp.inf), axis=0, keepdims=True)
        acc_ref[s:s + 1, :] = jnp.maximum(acc_ref[s:s + 1, :], my)

    @pl.when(g == (N // R) - 1)
    def _():
        t = _dot(acc_ref[...], m1w_ref[...]) + m1b_ref[...]
        t = _dot(t, m2w_ref[...]) + m2b_ref[...]
        out_ref[...] = _dot(t, hw_ref[...]) + hb_ref[...]


def _l1(x1, bg2, bat_col, c2_W, c2_b, l1_W, l1_b, m1_W, m1_b, m2_W, m2_b,
        h_W, h_b):
    full = lambda r, c: pl.BlockSpec((r, c), lambda i: (0, 0))
    return pl.pallas_call(
        _l1_body,
        grid=(N // R,),
        in_specs=[
            pl.BlockSpec((R, 64), lambda i: (i, 0)),
            pl.BlockSpec((R, KNN, 128), lambda i: (i, 0, 0)),
            pl.BlockSpec((R, 1), lambda i: (i, 0)),
            full(128, 128), full(1, 128),
            full(192, 1024), full(1, 1024),
            full(1024, 512), full(1, 512),
            full(512, 256), full(1, 256),
            full(256, 40), full(1, 40),
        ],
        out_specs=full(NG, 40),
        out_shape=jax.ShapeDtypeStruct((NG, 40), jnp.float32),
        scratch_shapes=[pltpu.VMEM((NG, 1024), jnp.float32)],
    )(x1, bg2, bat_col, c2_W, _vec(c2_b), l1_W, _vec(l1_b), m1_W,
      _vec(m1_b), m2_W, _vec(m2_b), h_W, _vec(h_b))


def kernel(pos, batch, c1_W1, c1_b1, c1_g1, c1_be1, c1_W2, c1_b2, c1_g2,
           c1_be2, c1_W3, c1_b3, c2_W, c2_b, l1_W, l1_b, m1_W, m1_b,
           m2_W, m2_b, h_W, h_b):
    bat = batch.astype(jnp.int32)
    bat_col = jnp.reshape(bat, (N, 1))
    bat_row = jnp.reshape(bat, (1, N))

    posp = jnp.concatenate([pos, jnp.zeros((N, 5), jnp.float32)], axis=1)
    post = jnp.concatenate([pos, jnp.zeros((N, 125), jnp.float32)], axis=1)
    zpad = jnp.zeros((5, 64), jnp.float32)
    wa = jnp.concatenate([c1_W1[0:3, :], zpad], axis=0)
    wb = jnp.concatenate([c1_W1[3:6, :], zpad], axis=0)

    sqp = _sqnorm(posp, 8)
    idx1 = _knn(posp, sqp, bat_col, jnp.reshape(sqp, (1, N)), bat_row, 8)
    idx1f = jnp.reshape(idx1[:, :KNN], (E,))
    bg1 = jnp.reshape(_gather_rows(post, idx1f, 128, 256), (N, KNN, 128))

    x1, sq1 = _conv1(posp, bg1, wa, wb, _vec(c1_b1), _vec(c1_g1),
                     _vec(c1_be1), c1_W2, _vec(c1_b2), _vec(c1_g2),
                     _vec(c1_be2), c1_W3, _vec(c1_b3))

    x1t = jnp.concatenate([x1, jnp.zeros((N, 64), jnp.float32)], axis=1)
    idx2 = _knn(x1, sq1, bat_col, jnp.reshape(sq1, (1, N)), bat_row, 64)
    idx2f = jnp.reshape(idx2[:, :KNN], (E,))
    bg2 = jnp.reshape(_gather_rows(x1t, idx2f, 128, 256), (N, KNN, 128))

    return _l1(x1, bg2, bat_col, c2_W, c2_b, l1_W, l1_b, m1_W, m1_b,
               m2_W, m2_b, h_W, h_b)


# speedup vs baseline: 13.8664x; 2.4693x over previous
import functools

import jax
import jax.numpy as jnp
from jax import lax
from jax.experimental import pallas as pl
from jax.experimental.pallas import tpu as pltpu
from jax.experimental.pallas import tpu_sc as plsc

N = 8192
KNN = 20
KPAD = 32
NG = 16
R = 256
E = N * KNN
EPS = 1e-5


def _bf(x):
    return x.astype(jnp.bfloat16)


def _dot(a, b):
    return jnp.dot(_bf(a), _bf(b), preferred_element_type=jnp.float32)


def _vec(v):
    return jnp.reshape(v, (1, -1))


def _sq_body(x_ref, sq_ref):
    x = x_ref[...]
    sq_ref[...] = jnp.sum(x * x, axis=1, keepdims=True)


def _sqnorm(x, F):
    return pl.pallas_call(
        _sq_body,
        grid=(N // R,),
        in_specs=[pl.BlockSpec((R, F), lambda i: (i, 0))],
        out_specs=pl.BlockSpec((R, 1), lambda i: (i, 0)),
        out_shape=jax.ShapeDtypeStruct((N, 1), jnp.float32),
    )(x)


W = 512
NCH = N // W


def _knn_body(xr_ref, sqr_ref, br_ref, xf_ref, bf_ref, sqc_ref, batc_ref,
              idx_ref):
    i0 = pl.program_id(0) * R
    xr = _bf(xr_ref[...])
    sqr = sqr_ref[...]
    br = br_ref[...]
    bf = bf_ref[...]
    b_lo = jnp.min(br)
    b_hi = jnp.max(br)
    cid = lax.broadcasted_iota(jnp.int32, (1, N), 1)
    col_lo = jnp.min(jnp.where(bf == b_lo, cid, N))
    col_hi = jnp.max(jnp.where(bf == b_hi, cid, 0))
    c0 = col_lo // W
    c1 = col_hi // W + 1
    rowid = i0 + lax.broadcasted_iota(jnp.int32, (R, W), 0)
    wiota = lax.broadcasted_iota(jnp.int32, (R, W), 1)
    pos_big = jnp.int32(N)

    def body(c, carry):
        vals, idxs = carry
        cstart = c * W
        xfc = _bf(xf_ref[pl.ds(cstart, W), :])
        sqfc = sqc_ref[c]
        bfc = batc_ref[c]
        d = sqfc + sqr - 2.0 * lax.dot_general(
            xr, xfc, (((1,), (1,)), ((), ())),
            preferred_element_type=jnp.float32)
        colc = cstart + wiota
        ok = (br == bfc) & (colc != rowid)
        d = jnp.where(ok, d, 1e37)
        cv = jnp.concatenate([vals, d], axis=1)
        ci = jnp.concatenate([idxs, colc], axis=1)
        nv, ni = [], []
        for _t in range(KNN):
            m = jnp.min(cv, axis=1, keepdims=True)
            am = jnp.min(jnp.where(cv <= m, ci, pos_big), axis=1,
                         keepdims=True)
            nv.append(m)
            ni.append(am)
            cv = jnp.where(ci == am, 2e38, cv)
        return jnp.concatenate(nv, axis=1), jnp.concatenate(ni, axis=1)

    vals0 = jnp.full((R, KNN), 2e38, jnp.float32)
    idxs0 = jnp.full((R, KNN), N, jnp.int32)
    _, idxs = lax.fori_loop(c0, c1, body, (vals0, idxs0))
    idxs = jnp.minimum(idxs, N - 1)
    idx_ref[...] = jnp.concatenate([idxs, idxs[:, 0:KPAD - KNN]], axis=1)


def _knn(x, sq_col, bat_col, sq_row, bat_row, F):
    sq_ch = jnp.reshape(sq_row, (NCH, 1, W))
    bat_ch = jnp.reshape(bat_row, (NCH, 1, W))
    return pl.pallas_call(
        _knn_body,
        grid=(N // R,),
        in_specs=[
            pl.BlockSpec((R, F), lambda i: (i, 0)),
            pl.BlockSpec((R, 1), lambda i: (i, 0)),
            pl.BlockSpec((R, 1), lambda i: (i, 0)),
            pl.BlockSpec((N, F), lambda i: (0, 0)),
            pl.BlockSpec((1, N), lambda i: (0, 0)),
            pl.BlockSpec((NCH, 1, W), lambda i: (0, 0, 0)),
            pl.BlockSpec((NCH, 1, W), lambda i: (0, 0, 0)),
        ],
        out_specs=pl.BlockSpec((R, KPAD), lambda i: (i, 0)),
        out_shape=jax.ShapeDtypeStruct((N, KPAD), jnp.int32),
        compiler_params=pltpu.CompilerParams(
            dimension_semantics=("parallel",)),
    )(x, sq_col, bat_col, x, bat_row, sq_ch, bat_ch)


def _gather_rows(table, idx, D, C):
    info = plsc.get_sparse_core_info()
    nw = info.num_cores * info.num_subcores
    bpw = E // nw
    nchunk = bpw // C
    mesh = plsc.VectorSubcoreMesh(core_axis_name="c", subcore_axis_name="s")

    @functools.partial(
        pl.kernel, mesh=mesh,
        out_type=jax.ShapeDtypeStruct((E, D), jnp.float32),
        scratch_types=[
            pltpu.VMEM((C,), jnp.int32),
            pltpu.VMEM((C, D), jnp.float32),
            pltpu.SemaphoreType.DMA,
        ],
    )
    def k(table_hbm, idx_hbm, out_hbm, idx_v, rows_v, sem):
        wid = lax.axis_index("s") * info.num_cores + lax.axis_index("c")
        base = wid * bpw
        for c in range(nchunk):
            off = base + c * C
            pltpu.sync_copy(idx_hbm.at[pl.ds(off, C)], idx_v)
            pltpu.async_copy(table_hbm.at[idx_v], rows_v, sem).wait()
            pltpu.sync_copy(rows_v, out_hbm.at[pl.ds(off, C)])

    return k(table, idx)


def _h1(pos_ref, bg_ref, wa_ref, wb_ref, b1_ref):
    xi = pos_ref[...]
    xia = _dot(xi, wa_ref[...]) + b1_ref[...]
    dx = bg_ref[...][:, :, 0:8] - xi[:, None, :]
    hb = _dot(jnp.reshape(dx, (R * KNN, 8)), wb_ref[...])
    return jnp.reshape(xia[:, None, :] + jnp.reshape(hb, (R, KNN, 64)),
                       (R * KNN, 64))


def _stats_from(st, cnt):
    mu = st[0:1, :] / cnt
    var = st[1:2, :] / cnt - mu * mu
    inv = lax.rsqrt(var + EPS)
    return mu, inv


def _acc_stats(st_ref, h):
    s = jnp.sum(h, axis=0, keepdims=True)
    s2 = jnp.sum(h * h, axis=0, keepdims=True)

    @pl.when(pl.program_id(0) == 0)
    def _():
        st_ref[...] = jnp.zeros_like(st_ref)

    st_ref[...] += jnp.concatenate([s, s2], axis=0)


def _e1_body(pos_ref, bg_ref, wa_ref, wb_ref, b1_ref, st_ref):
    _acc_stats(st_ref, _h1(pos_ref, bg_ref, wa_ref, wb_ref, b1_ref))


def _e2_body(pos_ref, bg_ref, wa_ref, wb_ref, b1_ref, st1_ref, g1_ref,
             be1_ref, w2_ref, b2_ref, st_ref):
    h = _h1(pos_ref, bg_ref, wa_ref, wb_ref, b1_ref)
    mu, inv = _stats_from(st1_ref[...], float(E))
    h = jnp.maximum((h - mu) * inv * g1_ref[...] + be1_ref[...], 0.0)
    h2 = _dot(h, w2_ref[...]) + b2_ref[...]
    _acc_stats(st_ref, h2)


def _e3_body(pos_ref, bg_ref, wa_ref, wb_ref, b1_ref, st1_ref, g1_ref,
             be1_ref, w2_ref, b2_ref, st2_ref, g2_ref, be2_ref, w3_ref,
             b3_ref, x1_ref, sq_ref):
    h = _h1(pos_ref, bg_ref, wa_ref, wb_ref, b1_ref)
    mu1, inv1 = _stats_from(st1_ref[...], float(E))
    h = jnp.maximum((h - mu1) * inv1 * g1_ref[...] + be1_ref[...], 0.0)
    h = _dot(h, w2_ref[...]) + b2_ref[...]
    mu2, inv2 = _stats_from(st2_ref[...], float(E))
    h = jnp.maximum((h - mu2) * inv2 * g2_ref[...] + be2_ref[...], 0.0)
    h = _dot(h, w3_ref[...]) + b3_ref[...]
    x1 = jnp.max(jnp.reshape(h, (R, KNN, 64)), axis=1)
    x1_ref[...] = x1
    sq_ref[...] = jnp.sum(x1 * x1, axis=1, keepdims=True)


def _conv1(posp, bg1, wa, wb, b1, g1, be1, w2, b2, g2, be2, w3, b3):
    grid = (N // R,)
    pos_spec = pl.BlockSpec((R, 8), lambda i: (i, 0))
    bg_spec = pl.BlockSpec((R, KNN, 128), lambda i: (i, 0, 0))
    st_spec = pl.BlockSpec((2, 64), lambda i: (0, 0))
    w8_spec = pl.BlockSpec((8, 64), lambda i: (0, 0))
    w_spec = pl.BlockSpec((64, 64), lambda i: (0, 0))
    v_spec = pl.BlockSpec((1, 64), lambda i: (0, 0))
    st_shape = jax.ShapeDtypeStruct((2, 64), jnp.float32)

    st1 = pl.pallas_call(
        _e1_body, grid=grid,
        in_specs=[pos_spec, bg_spec, w8_spec, w8_spec, v_spec],
        out_specs=st_spec, out_shape=st_shape)(posp, bg1, wa, wb, b1)
    st2 = pl.pallas_call(
        _e2_body, grid=grid,
        in_specs=[pos_spec, bg_spec, w8_spec, w8_spec, v_spec,
                  st_spec, v_spec, v_spec, w_spec, v_spec],
        out_specs=st_spec, out_shape=st_shape)(
            posp, bg1, wa, wb, b1, st1, g1, be1, w2, b2)
    x1, sq1 = pl.pallas_call(
        _e3_body, grid=grid,
        in_specs=[pos_spec, bg_spec, w8_spec, w8_spec, v_spec,
                  st_spec, v_spec, v_spec, w_spec, v_spec,
                  st_spec, v_spec, v_spec, w_spec, v_spec],
        out_specs=[pl.BlockSpec((R, 64), lambda i: (i, 0)),
                   pl.BlockSpec((R, 1), lambda i: (i, 0))],
        out_shape=[jax.ShapeDtypeStruct((N, 64), jnp.float32),
                   jax.ShapeDtypeStruct((N, 1), jnp.float32)])(
            posp, bg1, wa, wb, b1, st1, g1, be1, w2, b2,
            st2, g2, be2, w3, b3)
    return x1, sq1


def _l1_body(x1_ref, bg2_ref, bat_ref, c2w_ref, c2b_ref, lw_ref, lb_ref,
             m1w_ref, m1b_ref, m2w_ref, m2b_ref, hw_ref, hb_ref,
             out_ref, acc_ref):
    g = pl.program_id(0)
    x1 = x1_ref[...]
    xia = _dot(x1, c2w_ref[0:64, :]) + c2b_ref[...]
    dx = bg2_ref[...][:, :, 0:64] - x1[:, None, :]
    hb = _dot(jnp.reshape(dx, (R * KNN, 64)), c2w_ref[64:128, :])
    x2 = xia + jnp.max(jnp.reshape(hb, (R, KNN, 128)), axis=1)
    y = (_dot(x1, lw_ref[0:64, :]) + _dot(x2, lw_ref[64:192, :])
         + lb_ref[...])

    @pl.when(g == 0)
    def _():
        acc_ref[...] = jnp.full_like(acc_ref, -jnp.inf)

    bat = bat_ref[...]
    for s in range(NG):
        my = jnp.max(jnp.where(bat == s, y, -jnp.inf), axis=0, keepdims=True)
        acc_ref[s:s + 1, :] = jnp.maximum(acc_ref[s:s + 1, :], my)

    @pl.when(g == (N // R) - 1)
    def _():
        t = _dot(acc_ref[...], m1w_ref[...]) + m1b_ref[...]
        t = _dot(t, m2w_ref[...]) + m2b_ref[...]
        out_ref[...] = _dot(t, hw_ref[...]) + hb_ref[...]


def _l1(x1, bg2, bat_col, c2_W, c2_b, l1_W, l1_b, m1_W, m1_b, m2_W, m2_b,
        h_W, h_b):
    full = lambda r, c: pl.BlockSpec((r, c), lambda i: (0, 0))
    return pl.pallas_call(
        _l1_body,
        grid=(N // R,),
        in_specs=[
            pl.BlockSpec((R, 64), lambda i: (i, 0)),
            pl.BlockSpec((R, KNN, 128), lambda i: (i, 0, 0)),
            pl.BlockSpec((R, 1), lambda i: (i, 0)),
            full(128, 128), full(1, 128),
            full(192, 1024), full(1, 1024),
            full(1024, 512), full(1, 512),
            full(512, 256), full(1, 256),
            full(256, 40), full(1, 40),
        ],
        out_specs=full(NG, 40),
        out_shape=jax.ShapeDtypeStruct((NG, 40), jnp.float32),
        scratch_shapes=[pltpu.VMEM((NG, 1024), jnp.float32)],
    )(x1, bg2, bat_col, c2_W, _vec(c2_b), l1_W, _vec(l1_b), m1_W,
      _vec(m1_b), m2_W, _vec(m2_b), h_W, _vec(h_b))


def kernel(pos, batch, c1_W1, c1_b1, c1_g1, c1_be1, c1_W2, c1_b2, c1_g2,
           c1_be2, c1_W3, c1_b3, c2_W, c2_b, l1_W, l1_b, m1_W, m1_b,
           m2_W, m2_b, h_W, h_b):
    bat = batch.astype(jnp.int32)
    bat_col = jnp.reshape(bat, (N, 1))
    bat_row = jnp.reshape(bat, (1, N))

    posp = jnp.concatenate([pos, jnp.zeros((N, 5), jnp.float32)], axis=1)
    post = jnp.concatenate([pos, jnp.zeros((N, 125), jnp.float32)], axis=1)
    zpad = jnp.zeros((5, 64), jnp.float32)
    wa = jnp.concatenate([c1_W1[0:3, :], zpad], axis=0)
    wb = jnp.concatenate([c1_W1[3:6, :], zpad], axis=0)

    sqp = _sqnorm(posp, 8)
    idx1 = _knn(posp, sqp, bat_col, jnp.reshape(sqp, (1, N)), bat_row, 8)
    idx1f = jnp.reshape(idx1[:, :KNN], (E,))
    bg1 = jnp.reshape(_gather_rows(post, idx1f, 128, 256), (N, KNN, 128))

    x1, sq1 = _conv1(posp, bg1, wa, wb, _vec(c1_b1), _vec(c1_g1),
                     _vec(c1_be1), c1_W2, _vec(c1_b2), _vec(c1_g2),
                     _vec(c1_be2), c1_W3, _vec(c1_b3))

    x1t = jnp.concatenate([x1, jnp.zeros((N, 64), jnp.float32)], axis=1)
    idx2 = _knn(x1, sq1, bat_col, jnp.reshape(sq1, (1, N)), bat_row, 64)
    idx2f = jnp.reshape(idx2[:, :KNN], (E,))
    bg2 = jnp.reshape(_gather_rows(x1t, idx2f, 128, 256), (N, KNN, 128))

    return _l1(x1, bg2, bat_col, c2_W, c2_b, l1_W, l1_b, m1_W, m1_b,
               m2_W, m2_b, h_W, h_b)


# speedup vs baseline: 14.0211x; 1.0112x over previous
import functools

import jax
import jax.numpy as jnp
from jax import lax
from jax.experimental import pallas as pl
from jax.experimental.pallas import tpu as pltpu
from jax.experimental.pallas import tpu_sc as plsc

N = 8192
KNN = 20
KPAD = 32
NG = 16
R = 256
E = N * KNN
EPS = 1e-5


def _bf(x):
    return x.astype(jnp.bfloat16)


def _dot(a, b):
    return jnp.dot(_bf(a), _bf(b), preferred_element_type=jnp.float32)


def _vec(v):
    return jnp.reshape(v, (1, -1))


def _sq_body(x_ref, sq_ref):
    x = x_ref[...]
    sq_ref[...] = jnp.sum(x * x, axis=1, keepdims=True)


def _sqnorm(x, F):
    return pl.pallas_call(
        _sq_body,
        grid=(N // R,),
        in_specs=[pl.BlockSpec((R, F), lambda i: (i, 0))],
        out_specs=pl.BlockSpec((R, 1), lambda i: (i, 0)),
        out_shape=jax.ShapeDtypeStruct((N, 1), jnp.float32),
    )(x)


W = 512
NCH = N // W


def _knn_body(xr_ref, sqr_ref, br_ref, xf_ref, bf_ref, sqc_ref, batc_ref,
              idx_ref):
    i0 = pl.program_id(0) * R
    xr = _bf(xr_ref[...])
    sqr = sqr_ref[...]
    br = br_ref[...]
    bf = bf_ref[...]
    b_lo = jnp.min(br)
    b_hi = jnp.max(br)
    cid = lax.broadcasted_iota(jnp.int32, (1, N), 1)
    col_lo = jnp.min(jnp.where(bf == b_lo, cid, N))
    col_hi = jnp.max(jnp.where(bf == b_hi, cid, 0))
    c0 = col_lo // W
    c1 = col_hi // W + 1
    rowid = i0 + lax.broadcasted_iota(jnp.int32, (R, W), 0)
    wiota = lax.broadcasted_iota(jnp.int32, (R, W), 1)
    pos_big = jnp.int32(N)

    def body(c, carry):
        vals, idxs = carry
        cstart = c * W
        xfc = _bf(xf_ref[pl.ds(cstart, W), :])
        sqfc = sqc_ref[c]
        bfc = batc_ref[c]
        d = sqfc + sqr - 2.0 * lax.dot_general(
            xr, xfc, (((1,), (1,)), ((), ())),
            preferred_element_type=jnp.float32)
        colc = cstart + wiota
        ok = (br == bfc) & (colc != rowid)
        d = jnp.where(ok, d, 1e37)
        cv = jnp.concatenate([vals, d], axis=1)
        ci = jnp.concatenate([idxs, colc], axis=1)
        nv, ni = [], []
        for _t in range(KNN):
            m = jnp.min(cv, axis=1, keepdims=True)
            am = jnp.min(jnp.where(cv <= m, ci, pos_big), axis=1,
                         keepdims=True)
            nv.append(m)
            ni.append(am)
            cv = jnp.where(ci == am, 2e38, cv)
        return jnp.concatenate(nv, axis=1), jnp.concatenate(ni, axis=1)

    vals0 = jnp.full((R, KNN), 2e38, jnp.float32)
    idxs0 = jnp.full((R, KNN), N, jnp.int32)
    _, idxs = lax.fori_loop(c0, c1, body, (vals0, idxs0))
    idxs = jnp.minimum(idxs, N - 1)
    idx_ref[...] = jnp.concatenate([idxs, idxs[:, 0:KPAD - KNN]], axis=1)


def _knn(x, sq_col, bat_col, sq_row, bat_row, F):
    sq_ch = jnp.reshape(sq_row, (NCH, 1, W))
    bat_ch = jnp.reshape(bat_row, (NCH, 1, W))
    return pl.pallas_call(
        _knn_body,
        grid=(N // R,),
        in_specs=[
            pl.BlockSpec((R, F), lambda i: (i, 0)),
            pl.BlockSpec((R, 1), lambda i: (i, 0)),
            pl.BlockSpec((R, 1), lambda i: (i, 0)),
            pl.BlockSpec((N, F), lambda i: (0, 0)),
            pl.BlockSpec((1, N), lambda i: (0, 0)),
            pl.BlockSpec((NCH, 1, W), lambda i: (0, 0, 0)),
            pl.BlockSpec((NCH, 1, W), lambda i: (0, 0, 0)),
        ],
        out_specs=pl.BlockSpec((R, KPAD), lambda i: (i, 0)),
        out_shape=jax.ShapeDtypeStruct((N, KPAD), jnp.int32),
        compiler_params=pltpu.CompilerParams(
            dimension_semantics=("parallel",)),
    )(x, sq_col, bat_col, x, bat_row, sq_ch, bat_ch)


def _gather_rows(table, idx, D, C):
    info = plsc.get_sparse_core_info()
    nw = info.num_cores * info.num_subcores
    bpw = E // nw
    nchunk = bpw // C
    mesh = plsc.VectorSubcoreMesh(core_axis_name="c", subcore_axis_name="s")

    @functools.partial(
        pl.kernel, mesh=mesh,
        out_type=jax.ShapeDtypeStruct((E, D), jnp.float32),
        scratch_types=[
            pltpu.VMEM((C,), jnp.int32),
            pltpu.VMEM((C, D), jnp.float32),
            pltpu.SemaphoreType.DMA,
        ],
    )
    def k(table_hbm, idx_hbm, out_hbm, idx_v, rows_v, sem):
        wid = lax.axis_index("s") * info.num_cores + lax.axis_index("c")
        base = wid * bpw
        for c in range(nchunk):
            off = base + c * C
            pltpu.sync_copy(idx_hbm.at[pl.ds(off, C)], idx_v)
            pltpu.async_copy(table_hbm.at[idx_v], rows_v, sem).wait()
            pltpu.sync_copy(rows_v, out_hbm.at[pl.ds(off, C)])

    return k(table, idx)


def _h1_from_dx(pos_ref, dx, wa_ref, wb_ref, b1_ref):
    xi = pos_ref[...]
    xia = _dot(xi, wa_ref[...]) + b1_ref[...]
    hb = _dot(jnp.reshape(dx, (R * KNN, 8)), wb_ref[...])
    return jnp.reshape(xia[:, None, :] + jnp.reshape(hb, (R, KNN, 64)),
                       (R * KNN, 64))


def _stats_from(st, cnt):
    mu = st[0:1, :] / cnt
    var = st[1:2, :] / cnt - mu * mu
    inv = lax.rsqrt(var + EPS)
    return mu, inv


def _acc_stats(st_ref, h):
    s = jnp.sum(h, axis=0, keepdims=True)
    s2 = jnp.sum(h * h, axis=0, keepdims=True)

    @pl.when(pl.program_id(0) == 0)
    def _():
        st_ref[...] = jnp.zeros_like(st_ref)

    st_ref[...] += jnp.concatenate([s, s2], axis=0)


def _e1_body(pos_ref, bg_ref, wa_ref, wb_ref, b1_ref, st_ref, dx_ref):
    dx = bg_ref[...][:, :, 0:8] - pos_ref[...][:, None, :]
    dx_ref[...] = dx
    _acc_stats(st_ref, _h1_from_dx(pos_ref, dx, wa_ref, wb_ref, b1_ref))


def _e2_body(pos_ref, dx_ref, wa_ref, wb_ref, b1_ref, st1_ref, g1_ref,
             be1_ref, w2_ref, b2_ref, st_ref):
    h = _h1_from_dx(pos_ref, dx_ref[...], wa_ref, wb_ref, b1_ref)
    mu, inv = _stats_from(st1_ref[...], float(E))
    h = jnp.maximum((h - mu) * inv * g1_ref[...] + be1_ref[...], 0.0)
    h2 = _dot(h, w2_ref[...]) + b2_ref[...]
    _acc_stats(st_ref, h2)


def _e3_body(pos_ref, dx_ref, wa_ref, wb_ref, b1_ref, st1_ref, g1_ref,
             be1_ref, w2_ref, b2_ref, st2_ref, g2_ref, be2_ref, w3_ref,
             b3_ref, x1_ref, sq_ref):
    h = _h1_from_dx(pos_ref, dx_ref[...], wa_ref, wb_ref, b1_ref)
    mu1, inv1 = _stats_from(st1_ref[...], float(E))
    h = jnp.maximum((h - mu1) * inv1 * g1_ref[...] + be1_ref[...], 0.0)
    h = _dot(h, w2_ref[...]) + b2_ref[...]
    mu2, inv2 = _stats_from(st2_ref[...], float(E))
    h = jnp.maximum((h - mu2) * inv2 * g2_ref[...] + be2_ref[...], 0.0)
    h = _dot(h, w3_ref[...]) + b3_ref[...]
    x1 = jnp.max(jnp.reshape(h, (R, KNN, 64)), axis=1)
    x1_ref[...] = x1
    sq_ref[...] = jnp.sum(x1 * x1, axis=1, keepdims=True)


def _conv1(posp, bg1, wa, wb, b1, g1, be1, w2, b2, g2, be2, w3, b3):
    grid = (N // R,)
    pos_spec = pl.BlockSpec((R, 8), lambda i: (i, 0))
    bg_spec = pl.BlockSpec((R, KNN, 128), lambda i: (i, 0, 0))
    st_spec = pl.BlockSpec((2, 64), lambda i: (0, 0))
    w8_spec = pl.BlockSpec((8, 64), lambda i: (0, 0))
    w_spec = pl.BlockSpec((64, 64), lambda i: (0, 0))
    v_spec = pl.BlockSpec((1, 64), lambda i: (0, 0))
    st_shape = jax.ShapeDtypeStruct((2, 64), jnp.float32)

    dx_spec = pl.BlockSpec((R, KNN, 8), lambda i: (i, 0, 0))

    st1, dx = pl.pallas_call(
        _e1_body, grid=grid,
        in_specs=[pos_spec, bg_spec, w8_spec, w8_spec, v_spec],
        out_specs=[st_spec, dx_spec],
        out_shape=[st_shape,
                   jax.ShapeDtypeStruct((N, KNN, 8), jnp.float32)])(
            posp, bg1, wa, wb, b1)
    st2 = pl.pallas_call(
        _e2_body, grid=grid,
        in_specs=[pos_spec, dx_spec, w8_spec, w8_spec, v_spec,
                  st_spec, v_spec, v_spec, w_spec, v_spec],
        out_specs=st_spec, out_shape=st_shape)(
            posp, dx, wa, wb, b1, st1, g1, be1, w2, b2)
    x1, sq1 = pl.pallas_call(
        _e3_body, grid=grid,
        in_specs=[pos_spec, dx_spec, w8_spec, w8_spec, v_spec,
                  st_spec, v_spec, v_spec, w_spec, v_spec,
                  st_spec, v_spec, v_spec, w_spec, v_spec],
        out_specs=[pl.BlockSpec((R, 64), lambda i: (i, 0)),
                   pl.BlockSpec((R, 1), lambda i: (i, 0))],
        out_shape=[jax.ShapeDtypeStruct((N, 64), jnp.float32),
                   jax.ShapeDtypeStruct((N, 1), jnp.float32)])(
            posp, dx, wa, wb, b1, st1, g1, be1, w2, b2,
            st2, g2, be2, w3, b3)
    return x1, sq1


def _l1_body(x1_ref, bg2_ref, bat_ref, c2w_ref, c2b_ref, lw_ref, lb_ref,
             m1w_ref, m1b_ref, m2w_ref, m2b_ref, hw_ref, hb_ref,
             out_ref, acc_ref):
    g = pl.program_id(0)
    x1 = x1_ref[...]
    xia = _dot(x1, c2w_ref[0:64, :]) + c2b_ref[...]
    dx = bg2_ref[...][:, :, 0:64] - x1[:, None, :]
    hb = _dot(jnp.reshape(dx, (R * KNN, 64)), c2w_ref[64:128, :])
    x2 = xia + jnp.max(jnp.reshape(hb, (R, KNN, 128)), axis=1)
    y = (_dot(x1, lw_ref[0:64, :]) + _dot(x2, lw_ref[64:192, :])
         + lb_ref[...])

    @pl.when(g == 0)
    def _():
        acc_ref[...] = jnp.full_like(acc_ref, -jnp.inf)

    bat = bat_ref[...]
    b_lo = jnp.min(bat)
    b_hi = jnp.max(bat)

    def seg_body(s, _):
        my = jnp.max(jnp.where(bat == s, y, -jnp.inf), axis=0, keepdims=True)
        acc_ref[pl.ds(s, 1), :] = jnp.maximum(acc_ref[pl.ds(s, 1), :], my)
        return 0

    lax.fori_loop(b_lo, b_hi + 1, seg_body, 0)

    @pl.when(g == (N // R) - 1)
    def _():
        t = _dot(acc_ref[...], m1w_ref[...]) + m1b_ref[...]
        t = _dot(t, m2w_ref[...]) + m2b_ref[...]
        out_ref[...] = _dot(t, hw_ref[...]) + hb_ref[...]


def _l1(x1, bg2, bat_col, c2_W, c2_b, l1_W, l1_b, m1_W, m1_b, m2_W, m2_b,
        h_W, h_b):
    full = lambda r, c: pl.BlockSpec((r, c), lambda i: (0, 0))
    return pl.pallas_call(
        _l1_body,
        grid=(N // R,),
        in_specs=[
            pl.BlockSpec((R, 64), lambda i: (i, 0)),
            pl.BlockSpec((R, KNN, 128), lambda i: (i, 0, 0)),
            pl.BlockSpec((R, 1), lambda i: (i, 0)),
            full(128, 128), full(1, 128),
            full(192, 1024), full(1, 1024),
            full(1024, 512), full(1, 512),
            full(512, 256), full(1, 256),
            full(256, 40), full(1, 40),
        ],
        out_specs=full(NG, 40),
        out_shape=jax.ShapeDtypeStruct((NG, 40), jnp.float32),
        scratch_shapes=[pltpu.VMEM((NG, 1024), jnp.float32)],
    )(x1, bg2, bat_col, c2_W, _vec(c2_b), l1_W, _vec(l1_b), m1_W,
      _vec(m1_b), m2_W, _vec(m2_b), h_W, _vec(h_b))


def kernel(pos, batch, c1_W1, c1_b1, c1_g1, c1_be1, c1_W2, c1_b2, c1_g2,
           c1_be2, c1_W3, c1_b3, c2_W, c2_b, l1_W, l1_b, m1_W, m1_b,
           m2_W, m2_b, h_W, h_b):
    bat = batch.astype(jnp.int32)
    bat_col = jnp.reshape(bat, (N, 1))
    bat_row = jnp.reshape(bat, (1, N))

    posp = jnp.concatenate([pos, jnp.zeros((N, 5), jnp.float32)], axis=1)
    post = jnp.concatenate([pos, jnp.zeros((N, 125), jnp.float32)], axis=1)
    zpad = jnp.zeros((5, 64), jnp.float32)
    wa = jnp.concatenate([c1_W1[0:3, :], zpad], axis=0)
    wb = jnp.concatenate([c1_W1[3:6, :], zpad], axis=0)

    sqp = _sqnorm(posp, 8)
    idx1 = _knn(posp, sqp, bat_col, jnp.reshape(sqp, (1, N)), bat_row, 8)
    idx1f = jnp.reshape(idx1[:, :KNN], (E,))
    bg1 = jnp.reshape(_gather_rows(post, idx1f, 128, 256), (N, KNN, 128))

    x1, sq1 = _conv1(posp, bg1, wa, wb, _vec(c1_b1), _vec(c1_g1),
                     _vec(c1_be1), c1_W2, _vec(c1_b2), _vec(c1_g2),
                     _vec(c1_be2), c1_W3, _vec(c1_b3))

    x1t = jnp.concatenate([x1, jnp.zeros((N, 64), jnp.float32)], axis=1)
    idx2 = _knn(x1, sq1, bat_col, jnp.reshape(sq1, (1, N)), bat_row, 64)
    idx2f = jnp.reshape(idx2[:, :KNN], (E,))
    bg2 = jnp.reshape(_gather_rows(x1t, idx2f, 128, 256), (N, KNN, 128))

    return _l1(x1, bg2, bat_col, c2_W, c2_b, l1_W, l1_b, m1_W, m1_b,
               m2_W, m2_b, h_W, h_b)


# speedup vs baseline: 14.5232x; 1.0358x over previous
import functools

import jax
import jax.numpy as jnp
from jax import lax
from jax.experimental import pallas as pl
from jax.experimental.pallas import tpu as pltpu
from jax.experimental.pallas import tpu_sc as plsc

N = 8192
KNN = 20
KPAD = 32
NG = 16
R = 256
E = N * KNN
EPS = 1e-5


def _bf(x):
    return x.astype(jnp.bfloat16)


def _dot(a, b):
    return jnp.dot(_bf(a), _bf(b), preferred_element_type=jnp.float32)


def _vec(v):
    return jnp.reshape(v, (1, -1))


def _sq_body(x_ref, sq_ref):
    x = x_ref[...]
    sq_ref[...] = jnp.sum(x * x, axis=1, keepdims=True)


def _sqnorm(x, F):
    return pl.pallas_call(
        _sq_body,
        grid=(N // R,),
        in_specs=[pl.BlockSpec((R, F), lambda i: (i, 0))],
        out_specs=pl.BlockSpec((R, 1), lambda i: (i, 0)),
        out_shape=jax.ShapeDtypeStruct((N, 1), jnp.float32),
    )(x)


W = 512
NCH = N // W


def _knn_body(xr_ref, sqr_ref, br_ref, xf_ref, bf_ref, sqc_ref, batc_ref,
              idx_ref):
    i0 = pl.program_id(0) * R
    xr = _bf(xr_ref[...])
    sqr = sqr_ref[...]
    br = br_ref[...]
    bf = bf_ref[...]
    b_lo = jnp.min(br)
    b_hi = jnp.max(br)
    cid = lax.broadcasted_iota(jnp.int32, (1, N), 1)
    col_lo = jnp.min(jnp.where(bf == b_lo, cid, N))
    col_hi = jnp.max(jnp.where(bf == b_hi, cid, 0))
    c0 = col_lo // W
    c1 = col_hi // W + 1
    rowid = i0 + lax.broadcasted_iota(jnp.int32, (R, W), 0)
    wiota = lax.broadcasted_iota(jnp.int32, (R, W), 1)
    pos_big = jnp.int32(N)

    def body(c, carry):
        vals, idxs = carry
        cstart = c * W
        xfc = _bf(xf_ref[pl.ds(cstart, W), :])
        sqfc = sqc_ref[c]
        bfc = batc_ref[c]
        d = sqfc + sqr - 2.0 * lax.dot_general(
            xr, xfc, (((1,), (1,)), ((), ())),
            preferred_element_type=jnp.float32)
        colc = cstart + wiota
        ok = (br == bfc) & (colc != rowid)
        d = jnp.where(ok, d, 1e37)
        cv = jnp.concatenate([vals, d], axis=1)
        ci = jnp.concatenate([idxs, colc], axis=1)
        nv, ni = [], []
        for _t in range(KNN):
            m = jnp.min(cv, axis=1, keepdims=True)
            am = jnp.min(jnp.where(cv <= m, ci, pos_big), axis=1,
                         keepdims=True)
            nv.append(m)
            ni.append(am)
            cv = jnp.where(ci == am, 2e38, cv)
        return jnp.concatenate(nv, axis=1), jnp.concatenate(ni, axis=1)

    vals0 = jnp.full((R, KNN), 2e38, jnp.float32)
    idxs0 = jnp.full((R, KNN), N, jnp.int32)
    _, idxs = lax.fori_loop(c0, c1, body, (vals0, idxs0))
    idxs = jnp.minimum(idxs, N - 1)
    idx_ref[...] = jnp.concatenate([idxs, idxs[:, 0:KPAD - KNN]], axis=1)


def _knn(x, sq_col, bat_col, sq_row, bat_row, F):
    sq_ch = jnp.reshape(sq_row, (NCH, 1, W))
    bat_ch = jnp.reshape(bat_row, (NCH, 1, W))
    return pl.pallas_call(
        _knn_body,
        grid=(N // R,),
        in_specs=[
            pl.BlockSpec((R, F), lambda i: (i, 0)),
            pl.BlockSpec((R, 1), lambda i: (i, 0)),
            pl.BlockSpec((R, 1), lambda i: (i, 0)),
            pl.BlockSpec((N, F), lambda i: (0, 0)),
            pl.BlockSpec((1, N), lambda i: (0, 0)),
            pl.BlockSpec((NCH, 1, W), lambda i: (0, 0, 0)),
            pl.BlockSpec((NCH, 1, W), lambda i: (0, 0, 0)),
        ],
        out_specs=pl.BlockSpec((R, KPAD), lambda i: (i, 0)),
        out_shape=jax.ShapeDtypeStruct((N, KPAD), jnp.int32),
        compiler_params=pltpu.CompilerParams(
            dimension_semantics=("parallel",)),
    )(x, sq_col, bat_col, x, bat_row, sq_ch, bat_ch)


def _gather_rows(table, idx, D, C):
    info = plsc.get_sparse_core_info()
    nw = info.num_cores * info.num_subcores
    bpw = E // nw
    nchunk = bpw // C
    mesh = plsc.VectorSubcoreMesh(core_axis_name="c", subcore_axis_name="s")

    @functools.partial(
        pl.kernel, mesh=mesh,
        out_type=jax.ShapeDtypeStruct((E, D), jnp.float32),
        scratch_types=[
            pltpu.VMEM((C,), jnp.int32),
            pltpu.VMEM((C, D), jnp.float32),
            pltpu.SemaphoreType.DMA,
        ],
    )
    def k(table_hbm, idx_hbm, out_hbm, idx_v, rows_v, sem):
        wid = lax.axis_index("s") * info.num_cores + lax.axis_index("c")
        base = wid * bpw
        for c in range(nchunk):
            off = base + c * C
            pltpu.sync_copy(idx_hbm.at[pl.ds(off, C)], idx_v)
            pltpu.async_copy(table_hbm.at[idx_v], rows_v, sem).wait()
            pltpu.sync_copy(rows_v, out_hbm.at[pl.ds(off, C)])

    return k(table, idx)


def _h1_from_dx(pos_ref, dx, wa_ref, wb_ref, b1_ref):
    xi = pos_ref[...]
    xia = _dot(xi, wa_ref[...]) + b1_ref[...]
    hb = _dot(jnp.reshape(dx, (R * KNN, 8)), wb_ref[...])
    return jnp.reshape(xia[:, None, :] + jnp.reshape(hb, (R, KNN, 64)),
                       (R * KNN, 64))


def _stats_from(st, cnt):
    mu = st[0:1, :] / cnt
    var = st[1:2, :] / cnt - mu * mu
    inv = lax.rsqrt(var + EPS)
    return mu, inv


def _acc_stats(st_ref, h):
    s = jnp.sum(h, axis=0, keepdims=True)
    s2 = jnp.sum(h * h, axis=0, keepdims=True)

    @pl.when(pl.program_id(0) == 0)
    def _():
        st_ref[...] = jnp.zeros_like(st_ref)

    st_ref[...] += jnp.concatenate([s, s2], axis=0)


def _e1_body(pos_ref, bg_ref, wa_ref, wb_ref, b1_ref, st_ref, h1_ref):
    dx = bg_ref[...][:, :, 0:8] - pos_ref[...][:, None, :]
    h = _h1_from_dx(pos_ref, dx, wa_ref, wb_ref, b1_ref)
    h1_ref[...] = h
    _acc_stats(st_ref, h)


def _e2_body(h1_ref, st1_ref, g1_ref, be1_ref, w2_ref, b2_ref,
             st_ref, h2_ref):
    h = h1_ref[...]
    mu, inv = _stats_from(st1_ref[...], float(E))
    h = jnp.maximum((h - mu) * inv * g1_ref[...] + be1_ref[...], 0.0)
    h2 = _dot(h, w2_ref[...]) + b2_ref[...]
    h2_ref[...] = h2
    _acc_stats(st_ref, h2)


def _e3_body(h2_ref, st2_ref, g2_ref, be2_ref, w3_ref,
             b3_ref, x1_ref, sq_ref):
    h = h2_ref[...]
    mu2, inv2 = _stats_from(st2_ref[...], float(E))
    h = jnp.maximum((h - mu2) * inv2 * g2_ref[...] + be2_ref[...], 0.0)
    h = _dot(h, w3_ref[...]) + b3_ref[...]
    x1 = jnp.max(jnp.reshape(h, (R, KNN, 64)), axis=1)
    x1_ref[...] = x1
    sq_ref[...] = jnp.sum(x1 * x1, axis=1, keepdims=True)


def _conv1(posp, bg1, wa, wb, b1, g1, be1, w2, b2, g2, be2, w3, b3):
    grid = (N // R,)
    pos_spec = pl.BlockSpec((R, 8), lambda i: (i, 0))
    bg_spec = pl.BlockSpec((R, KNN, 128), lambda i: (i, 0, 0))
    st_spec = pl.BlockSpec((2, 64), lambda i: (0, 0))
    w8_spec = pl.BlockSpec((8, 64), lambda i: (0, 0))
    w_spec = pl.BlockSpec((64, 64), lambda i: (0, 0))
    v_spec = pl.BlockSpec((1, 64), lambda i: (0, 0))
    st_shape = jax.ShapeDtypeStruct((2, 64), jnp.float32)

    h_spec = pl.BlockSpec((R * KNN, 64), lambda i: (i, 0))
    h_shape = jax.ShapeDtypeStruct((E, 64), jnp.float32)

    st1, h1 = pl.pallas_call(
        _e1_body, grid=grid,
        in_specs=[pos_spec, bg_spec, w8_spec, w8_spec, v_spec],
        out_specs=[st_spec, h_spec],
        out_shape=[st_shape, h_shape])(posp, bg1, wa, wb, b1)
    st2, h2 = pl.pallas_call(
        _e2_body, grid=grid,
        in_specs=[h_spec, st_spec, v_spec, v_spec, w_spec, v_spec],
        out_specs=[st_spec, h_spec],
        out_shape=[st_shape, h_shape])(h1, st1, g1, be1, w2, b2)
    x1, sq1 = pl.pallas_call(
        _e3_body, grid=grid,
        in_specs=[h_spec, st_spec, v_spec, v_spec, w_spec, v_spec],
        out_specs=[pl.BlockSpec((R, 64), lambda i: (i, 0)),
                   pl.BlockSpec((R, 1), lambda i: (i, 0))],
        out_shape=[jax.ShapeDtypeStruct((N, 64), jnp.float32),
                   jax.ShapeDtypeStruct((N, 1), jnp.float32)])(
            h2, st2, g2, be2, w3, b3)
    return x1, sq1


def _l1_body(x1_ref, bg2_ref, bat_ref, c2w_ref, c2b_ref, lw_ref, lb_ref,
             m1w_ref, m1b_ref, m2w_ref, m2b_ref, hw_ref, hb_ref,
             out_ref, acc_ref):
    g = pl.program_id(0)
    x1 = x1_ref[...]
    xia = _dot(x1, c2w_ref[0:64, :]) + c2b_ref[...]
    dx = bg2_ref[...][:, :, 0:64] - x1[:, None, :]
    hb = _dot(jnp.reshape(dx, (R * KNN, 64)), c2w_ref[64:128, :])
    x2 = xia + jnp.max(jnp.reshape(hb, (R, KNN, 128)), axis=1)
    y = (_dot(x1, lw_ref[0:64, :]) + _dot(x2, lw_ref[64:192, :])
         + lb_ref[...])

    @pl.when(g == 0)
    def _():
        acc_ref[...] = jnp.full_like(acc_ref, -jnp.inf)

    bat = bat_ref[...]
    b_lo = jnp.min(bat)
    b_hi = jnp.max(bat)

    def seg_body(s, _):
        my = jnp.max(jnp.where(bat == s, y, -jnp.inf), axis=0, keepdims=True)
        acc_ref[pl.ds(s, 1), :] = jnp.maximum(acc_ref[pl.ds(s, 1), :], my)
        return 0

    lax.fori_loop(b_lo, b_hi + 1, seg_body, 0)

    @pl.when(g == (N // R) - 1)
    def _():
        t = _dot(acc_ref[...], m1w_ref[...]) + m1b_ref[...]
        t = _dot(t, m2w_ref[...]) + m2b_ref[...]
        out_ref[...] = _dot(t, hw_ref[...]) + hb_ref[...]


def _l1(x1, bg2, bat_col, c2_W, c2_b, l1_W, l1_b, m1_W, m1_b, m2_W, m2_b,
        h_W, h_b):
    full = lambda r, c: pl.BlockSpec((r, c), lambda i: (0, 0))
    return pl.pallas_call(
        _l1_body,
        grid=(N // R,),
        in_specs=[
            pl.BlockSpec((R, 64), lambda i: (i, 0)),
            pl.BlockSpec((R, KNN, 128), lambda i: (i, 0, 0)),
            pl.BlockSpec((R, 1), lambda i: (i, 0)),
            full(128, 128), full(1, 128),
            full(192, 1024), full(1, 1024),
            full(1024, 512), full(1, 512),
            full(512, 256), full(1, 256),
            full(256, 40), full(1, 40),
        ],
        out_specs=full(NG, 40),
        out_shape=jax.ShapeDtypeStruct((NG, 40), jnp.float32),
        scratch_shapes=[pltpu.VMEM((NG, 1024), jnp.float32)],
    )(x1, bg2, bat_col, c2_W, _vec(c2_b), l1_W, _vec(l1_b), m1_W,
      _vec(m1_b), m2_W, _vec(m2_b), h_W, _vec(h_b))


def kernel(pos, batch, c1_W1, c1_b1, c1_g1, c1_be1, c1_W2, c1_b2, c1_g2,
           c1_be2, c1_W3, c1_b3, c2_W, c2_b, l1_W, l1_b, m1_W, m1_b,
           m2_W, m2_b, h_W, h_b):
    bat = batch.astype(jnp.int32)
    bat_col = jnp.reshape(bat, (N, 1))
    bat_row = jnp.reshape(bat, (1, N))

    posp = jnp.concatenate([pos, jnp.zeros((N, 5), jnp.float32)], axis=1)
    post = jnp.concatenate([pos, jnp.zeros((N, 125), jnp.float32)], axis=1)
    zpad = jnp.zeros((5, 64), jnp.float32)
    wa = jnp.concatenate([c1_W1[0:3, :], zpad], axis=0)
    wb = jnp.concatenate([c1_W1[3:6, :], zpad], axis=0)

    sqp = _sqnorm(posp, 8)
    idx1 = _knn(posp, sqp, bat_col, jnp.reshape(sqp, (1, N)), bat_row, 8)
    idx1f = jnp.reshape(idx1[:, :KNN], (E,))
    bg1 = jnp.reshape(_gather_rows(post, idx1f, 128, 256), (N, KNN, 128))

    x1, sq1 = _conv1(posp, bg1, wa, wb, _vec(c1_b1), _vec(c1_g1),
                     _vec(c1_be1), c1_W2, _vec(c1_b2), _vec(c1_g2),
                     _vec(c1_be2), c1_W3, _vec(c1_b3))

    x1t = jnp.concatenate([x1, jnp.zeros((N, 64), jnp.float32)], axis=1)
    idx2 = _knn(x1, sq1, bat_col, jnp.reshape(sq1, (1, N)), bat_row, 64)
    idx2f = jnp.reshape(idx2[:, :KNN], (E,))
    bg2 = jnp.reshape(_gather_rows(x1t, idx2f, 128, 256), (N, KNN, 128))

    return _l1(x1, bg2, bat_col, c2_W, c2_b, l1_W, l1_b, m1_W, m1_b,
               m2_W, m2_b, h_W, h_b)
